# Initial kernel scaffold; baseline (speedup 1.0000x reference)
import jax
import jax.numpy as jnp
from jax.experimental import pallas as pl


def kernel(node_attrs, edge_attrs, instruction_batch, distribution, node_prop_similarities, relation_similarity, weight_node_properties, weight_edge, weight_node_score, weight_relation_score, node_indices, edge_batch_indices, edge_indices):
    raise NotImplementedError("write your pallas kernel here")


# speedup vs baseline: 5.7520x; 5.7520x over previous
import functools

import jax
import jax.numpy as jnp
from jax import lax
from jax.experimental import pallas as pl
from jax.experimental.pallas import tpu as pltpu
from jax.experimental.pallas import tpu_sc as plsc

N = 10000
E = 320000
B = 64
P = 4
D = 128

EB = 4000
NB = 1000
NC = 2
NS = 16
NW = NC * NS
EPW = E // NW
CH = 80
NCH = EPW // CH


def _elu(t):
    return jnp.where(t > 0, t, jnp.exp(jnp.minimum(t, 0.0)) - 1.0)


def _sc_gather(src3, dist):
    mesh = plsc.VectorSubcoreMesh(core_axis_name="c", subcore_axis_name="s")

    @functools.partial(
        pl.kernel,
        mesh=mesh,
        out_type=jax.ShapeDtypeStruct((NW, EPW), jnp.float32),
        scratch_types=[
            pltpu.VMEM((NCH, CH), jnp.int32),
            pltpu.VMEM((EPW,), jnp.float32),
            pltpu.SemaphoreType.DMA,
        ],
    )
    def sc_k(src_hbm, dist_hbm, out_hbm, srci, dv, sem):
        wid = lax.axis_index("s") * NC + lax.axis_index("c")
        pltpu.sync_copy(src_hbm.at[wid], srci)

        def fire(j, carry):
            pltpu.async_copy(dist_hbm.at[srci.at[j]],
                             dv.at[pl.ds(j * CH, CH)], sem)
            return carry

        lax.fori_loop(0, NCH, fire, 0)

        def drain(j, carry):
            pltpu.make_async_copy(dist_hbm.at[srci.at[j]],
                                  dv.at[pl.ds(j * CH, CH)], sem).wait()
            return carry

        lax.fori_loop(0, NCH, drain, 0)
        pltpu.sync_copy(dv, out_hbm.at[wid])

    return sc_k(src3, dist)


def _edge_body(xe_ref, we_ref, instr_ref, d_ref, eb_ref, out_ref):
    z = jnp.dot(xe_ref[...].astype(jnp.bfloat16),
                we_ref[...].astype(jnp.bfloat16),
                preferred_element_type=jnp.float32)
    eb = eb_ref[0]
    iota = lax.broadcasted_iota(jnp.int32, (B, 1), 0)
    oh = (iota == eb).astype(jnp.float32)
    ir = lax.dot_general(oh, instr_ref[...],
                         (((0,), (0,)), ((), ())),
                         preferred_element_type=jnp.float32,
                         precision=lax.Precision.HIGHEST)
    out_ref[...] = d_ref[...] * _elu(ir * z)


def _edge_messages(edge_attrs, weT, instr, d2, eb3):
    return pl.pallas_call(
        _edge_body,
        grid=(E // EB,),
        in_specs=[
            pl.BlockSpec((EB, D), lambda i: (i, 0)),
            pl.BlockSpec((D, D), lambda i: (0, 0)),
            pl.BlockSpec((B, D), lambda i: (0, 0)),
            pl.BlockSpec((EB, 1), lambda i: (i, 0)),
            pl.BlockSpec((1, 1, EB), lambda i: (i, 0, 0)),
        ],
        out_specs=pl.BlockSpec((EB, D), lambda i: (i, 0)),
        out_shape=jax.ShapeDtypeStruct((E, D), jnp.float32),
    )(edge_attrs, weT, instr, d2, eb3)


def _sc_scatter(m, dst3, zeros):
    mesh = plsc.VectorSubcoreMesh(core_axis_name="c", subcore_axis_name="s")

    @functools.partial(
        pl.kernel,
        mesh=mesh,
        out_type=jax.ShapeDtypeStruct((NC, N, D), jnp.float32),
        scratch_types=[
            pltpu.VMEM((NCH, CH), jnp.int32),
            pltpu.VMEM((CH, D), jnp.float32),
            pltpu.VMEM_SHARED((N, D), jnp.float32),
            pltpu.SemaphoreType.DMA,
        ],
    )
    def sc_k(m_hbm, dst_hbm, zeros_hbm, out_hbm, dsti, mrow, acc, sem):
        c = lax.axis_index("c")
        sid = lax.axis_index("s")
        wid = sid * NC + c
        base = wid * EPW
        pltpu.sync_copy(dst_hbm.at[wid], dsti)

        @pl.when(sid == 0)
        def _():
            pltpu.sync_copy(zeros_hbm, acc)

        plsc.subcore_barrier()

        def body(k, carry):
            pltpu.sync_copy(m_hbm.at[pl.ds(base + k * CH, CH)], mrow)
            pltpu.sync_copy(mrow, acc.at[dsti.at[k]], add=True)
            return carry

        lax.fori_loop(0, NCH, body, 0)
        plsc.subcore_barrier()

        @pl.when(sid == 0)
        def _():
            pltpu.sync_copy(acc, out_hbm.at[c])

    return sc_k(m, dst3, zeros)


def _node_body(x_ref, wT_ref, nps_ref, instr_ref, wns_ref, g_ref, out_ref):
    g = g_ref[0]
    iota = lax.broadcasted_iota(jnp.int32, (B, 1), 0)
    oh = (iota == g).astype(jnp.float32)
    a = lax.dot_general(oh, nps_ref[...],
                        (((0,), (0,)), ((), ())),
                        preferred_element_type=jnp.float32,
                        precision=lax.Precision.HIGHEST)
    ir = lax.dot_general(oh, instr_ref[...],
                         (((0,), (0,)), ((), ())),
                         preferred_element_type=jnp.float32,
                         precision=lax.Precision.HIGHEST)
    acc = jnp.zeros((NB, D), jnp.float32)
    for p in range(P):
        zp = jnp.dot(x_ref[:, p * D:(p + 1) * D].astype(jnp.bfloat16),
                     wT_ref[p * D:(p + 1) * D, :].astype(jnp.bfloat16),
                     preferred_element_type=jnp.float32)
        acc = acc + (a[:, p:p + 1] * ir) * zp
    ns = _elu(acc)
    out_ref[0, 0, :] = jnp.dot(ns.astype(jnp.bfloat16),
                               wns_ref[...].astype(jnp.bfloat16).reshape(D),
                               preferred_element_type=jnp.float32)


def _state_logits(x2, wT, nps, instr, wns2, g3):
    o3 = pl.pallas_call(
        _node_body,
        grid=(N // NB,),
        in_specs=[
            pl.BlockSpec((NB, P * D), lambda i: (i, 0)),
            pl.BlockSpec((P * D, D), lambda i: (0, 0)),
            pl.BlockSpec((B, P), lambda i: (0, 0)),
            pl.BlockSpec((B, D), lambda i: (0, 0)),
            pl.BlockSpec((1, D), lambda i: (0, 0)),
            pl.BlockSpec((1, 1, NB), lambda i: (i, 0, 0)),
        ],
        out_specs=pl.BlockSpec((1, 1, NB), lambda i: (i, 0, 0)),
        out_shape=jax.ShapeDtypeStruct((N // NB, 1, NB), jnp.float32),
    )(x2, wT, nps, instr, wns2, g3)
    return o3.reshape(N)


def _final_body(sl_ref, acc_ref, wrs_ref, g_ref, rs_ref, out_ref):
    ap = acc_ref[0] + acc_ref[1]
    rel = jnp.dot(ap.astype(jnp.bfloat16),
                  wrs_ref[...].astype(jnp.bfloat16).reshape(D, 1),
                  preferred_element_type=jnp.float32)
    g = g_ref[...]
    iota = lax.broadcasted_iota(jnp.int32, (1, B), 1)
    gmask = g == iota
    sl = sl_ref[...]

    def seg_softmax(l):
        mx = jnp.max(jnp.where(gmask, l, -1e30), axis=0, keepdims=True)
        mxg = jnp.sum(jnp.where(gmask, mx, 0.0), axis=1, keepdims=True)
        ex = jnp.exp(l - mxg)
        sm = jnp.sum(jnp.where(gmask, ex, 0.0), axis=0, keepdims=True)
        smg = jnp.sum(jnp.where(gmask, sm, 0.0), axis=1, keepdims=True)
        return ex / smg

    ps = seg_softmax(sl)
    pr = seg_softmax(rel)
    rsg = jnp.sum(jnp.where(gmask, rs_ref[...], 0.0), axis=1, keepdims=True)
    out_ref[...] = rsg * pr + (1.0 - rsg) * ps


def _finalize(sl2, acc, wrs2, g2, rs2):
    return pl.pallas_call(
        _final_body,
        out_shape=jax.ShapeDtypeStruct((N, 1), jnp.float32),
    )(sl2, acc, wrs2, g2, rs2)


def kernel(node_attrs, edge_attrs, instruction_batch, distribution,
           node_prop_similarities, relation_similarity,
           weight_node_properties, weight_edge, weight_node_score,
           weight_relation_score, node_indices, edge_batch_indices,
           edge_indices):
    g = node_indices.astype(jnp.int32)
    eb = edge_batch_indices.astype(jnp.int32)
    src = edge_indices[0].astype(jnp.int32)
    dst = edge_indices[1].astype(jnp.int32)

    src3 = src.reshape(NW, NCH, CH)
    d = _sc_gather(src3, distribution)

    weT = weight_edge.T
    eb3 = eb.reshape(E // EB, 1, EB)
    d2 = d.reshape(E, 1)
    m = _edge_messages(edge_attrs, weT, instruction_batch, d2, eb3)

    dst3 = dst.reshape(NW, NCH, CH)
    zeros = jnp.zeros((N, D), jnp.float32)
    acc = _sc_scatter(m, dst3, zeros)

    x2 = node_attrs.reshape(N, P * D)
    wT = jnp.transpose(weight_node_properties, (0, 2, 1)).reshape(P * D, D)
    wns2 = weight_node_score.reshape(1, D)
    g3 = g.reshape(N // NB, 1, NB)
    sl = _state_logits(x2, wT, node_prop_similarities,
                       instruction_batch, wns2, g3)

    sl2 = sl.reshape(N, 1)
    g2 = g.reshape(N, 1)
    wrs2 = weight_relation_score.reshape(1, D)
    rs2 = relation_similarity.reshape(1, B)
    out = _finalize(sl2, acc, wrs2, g2, rs2)
    return out.reshape(N)


# speedup vs baseline: 6.5935x; 1.1463x over previous
import functools

import jax
import jax.numpy as jnp
from jax import lax
from jax.experimental import pallas as pl
from jax.experimental.pallas import tpu as pltpu
from jax.experimental.pallas import tpu_sc as plsc

N = 10000
E = 320000
B = 64
P = 4
D = 128

EB = 4000
NB = 1000
NC = 2
NS = 16
NW = NC * NS
EPW = E // NW
CH1 = 80
NCH1 = EPW // CH1
CH = 80
NCH = EPW // CH


def _elu(t):
    return jnp.where(t > 0, t, jnp.exp(jnp.minimum(t, 0.0)) - 1.0)


def _sc_gather(src3, dist):
    mesh = plsc.VectorSubcoreMesh(core_axis_name="c", subcore_axis_name="s")

    @functools.partial(
        pl.kernel,
        mesh=mesh,
        out_type=jax.ShapeDtypeStruct((NW, EPW), jnp.float32),
        scratch_types=[
            pltpu.VMEM((NCH1, CH1), jnp.int32),
            pltpu.VMEM((EPW,), jnp.float32),
            pltpu.SemaphoreType.DMA,
        ],
    )
    def sc_k(src_hbm, dist_hbm, out_hbm, srci, dv, sem):
        wid = lax.axis_index("s") * NC + lax.axis_index("c")
        pltpu.sync_copy(src_hbm.at[wid], srci)

        def fire(j, carry):
            pltpu.async_copy(dist_hbm.at[srci.at[j]],
                             dv.at[pl.ds(j * CH1, CH1)], sem)
            return carry

        lax.fori_loop(0, NCH1, fire, 0)

        def drain(j, carry):
            pltpu.make_async_copy(dist_hbm.at[srci.at[j]],
                                  dv.at[pl.ds(j * CH1, CH1)], sem).wait()
            return carry

        lax.fori_loop(0, NCH1, drain, 0)
        pltpu.sync_copy(dv, out_hbm.at[wid])

    return sc_k(src3, dist)


def _edge_body(xe_ref, we_ref, instr_ref, d_ref, eb_ref, out_ref):
    z = jnp.dot(xe_ref[...].astype(jnp.bfloat16),
                we_ref[...].astype(jnp.bfloat16),
                preferred_element_type=jnp.float32)
    eb = eb_ref[0]
    iota = lax.broadcasted_iota(jnp.int32, (B, 1), 0)
    oh = (iota == eb).astype(jnp.float32)
    ir = lax.dot_general(oh, instr_ref[...],
                         (((0,), (0,)), ((), ())),
                         preferred_element_type=jnp.float32,
                         precision=lax.Precision.HIGHEST)
    out_ref[...] = d_ref[...] * _elu(ir * z)


def _edge_messages(edge_attrs, weT, instr, d2, eb3):
    return pl.pallas_call(
        _edge_body,
        grid=(E // EB,),
        in_specs=[
            pl.BlockSpec((EB, D), lambda i: (i, 0)),
            pl.BlockSpec((D, D), lambda i: (0, 0)),
            pl.BlockSpec((B, D), lambda i: (0, 0)),
            pl.BlockSpec((EB, 1), lambda i: (i, 0)),
            pl.BlockSpec((1, 1, EB), lambda i: (i, 0, 0)),
        ],
        out_specs=pl.BlockSpec((EB, D), lambda i: (i, 0)),
        out_shape=jax.ShapeDtypeStruct((E, D), jnp.float32),
    )(edge_attrs, weT, instr, d2, eb3)


def _sc_scatter(m, dst3, zeros):
    mesh = plsc.VectorSubcoreMesh(core_axis_name="c", subcore_axis_name="s")

    @functools.partial(
        pl.kernel,
        mesh=mesh,
        out_type=jax.ShapeDtypeStruct((NC, N, D), jnp.float32),
        scratch_types=[
            pltpu.VMEM((NCH, CH), jnp.int32),
            pltpu.VMEM((2, CH, D), jnp.float32),
            pltpu.VMEM_SHARED((N, D), jnp.float32),
            pltpu.SemaphoreType.DMA,
        ],
    )
    def sc_k(m_hbm, dst_hbm, zeros_hbm, out_hbm, dsti, mrow, acc, sem):
        c = lax.axis_index("c")
        sid = lax.axis_index("s")
        wid = sid * NC + c
        base = wid * EPW
        pltpu.sync_copy(dst_hbm.at[wid], dsti)

        @pl.when(sid == 0)
        def _():
            pltpu.sync_copy(zeros_hbm, acc)

        plsc.subcore_barrier()

        for b in range(2):
            pltpu.async_copy(m_hbm.at[pl.ds(base + b * CH, CH)],
                             mrow.at[b], sem)

        def body(t, carry):
            for b in range(2):
                k = 2 * t + b
                pltpu.make_async_copy(m_hbm.at[pl.ds(base + k * CH, CH)],
                                      mrow.at[b], sem).wait()
                pltpu.sync_copy(mrow.at[b], acc.at[dsti.at[k]], add=True)

                @pl.when(k + 2 < NCH)
                def _():
                    pltpu.async_copy(
                        m_hbm.at[pl.ds(base + (k + 2) * CH, CH)],
                        mrow.at[b], sem)
            return carry

        lax.fori_loop(0, NCH // 2, body, 0)
        kt = NCH - 1
        pltpu.make_async_copy(m_hbm.at[pl.ds(base + kt * CH, CH)],
                              mrow.at[0], sem).wait()
        pltpu.sync_copy(mrow.at[0], acc.at[dsti.at[kt]], add=True)
        plsc.subcore_barrier()

        @pl.when(sid == 0)
        def _():
            pltpu.sync_copy(acc, out_hbm.at[c])

    return sc_k(m, dst3, zeros)


def _node_body(x_ref, wT_ref, nps_ref, instr_ref, wns_ref, g_ref, out_ref):
    g = g_ref[0]
    iota = lax.broadcasted_iota(jnp.int32, (B, 1), 0)
    oh = (iota == g).astype(jnp.float32)
    a = lax.dot_general(oh, nps_ref[...],
                        (((0,), (0,)), ((), ())),
                        preferred_element_type=jnp.float32,
                        precision=lax.Precision.HIGHEST)
    ir = lax.dot_general(oh, instr_ref[...],
                         (((0,), (0,)), ((), ())),
                         preferred_element_type=jnp.float32,
                         precision=lax.Precision.HIGHEST)
    acc = jnp.zeros((NB, D), jnp.float32)
    for p in range(P):
        zp = jnp.dot(x_ref[:, p * D:(p + 1) * D].astype(jnp.bfloat16),
                     wT_ref[p * D:(p + 1) * D, :].astype(jnp.bfloat16),
                     preferred_element_type=jnp.float32)
        acc = acc + (a[:, p:p + 1] * ir) * zp
    ns = _elu(acc)
    out_ref[0, 0, :] = jnp.dot(ns.astype(jnp.bfloat16),
                               wns_ref[...].astype(jnp.bfloat16).reshape(D),
                               preferred_element_type=jnp.float32)


def _state_logits(x2, wT, nps, instr, wns2, g3):
    o3 = pl.pallas_call(
        _node_body,
        grid=(N // NB,),
        in_specs=[
            pl.BlockSpec((NB, P * D), lambda i: (i, 0)),
            pl.BlockSpec((P * D, D), lambda i: (0, 0)),
            pl.BlockSpec((B, P), lambda i: (0, 0)),
            pl.BlockSpec((B, D), lambda i: (0, 0)),
            pl.BlockSpec((1, D), lambda i: (0, 0)),
            pl.BlockSpec((1, 1, NB), lambda i: (i, 0, 0)),
        ],
        out_specs=pl.BlockSpec((1, 1, NB), lambda i: (i, 0, 0)),
        out_shape=jax.ShapeDtypeStruct((N // NB, 1, NB), jnp.float32),
    )(x2, wT, nps, instr, wns2, g3)
    return o3.reshape(N)


def _final_body(sl_ref, acc_ref, wrs_ref, g_ref, rs_ref, out_ref):
    ap = acc_ref[0] + acc_ref[1]
    rel = jnp.dot(ap.astype(jnp.bfloat16),
                  wrs_ref[...].astype(jnp.bfloat16).reshape(D, 1),
                  preferred_element_type=jnp.float32)
    g = g_ref[...]
    iota = lax.broadcasted_iota(jnp.int32, (1, B), 1)
    gmask = g == iota
    sl = sl_ref[...]

    def seg_softmax(l):
        mx = jnp.max(jnp.where(gmask, l, -1e30), axis=0, keepdims=True)
        mxg = jnp.sum(jnp.where(gmask, mx, 0.0), axis=1, keepdims=True)
        ex = jnp.exp(l - mxg)
        sm = jnp.sum(jnp.where(gmask, ex, 0.0), axis=0, keepdims=True)
        smg = jnp.sum(jnp.where(gmask, sm, 0.0), axis=1, keepdims=True)
        return ex / smg

    ps = seg_softmax(sl)
    pr = seg_softmax(rel)
    rsg = jnp.sum(jnp.where(gmask, rs_ref[...], 0.0), axis=1, keepdims=True)
    out_ref[...] = rsg * pr + (1.0 - rsg) * ps


def _finalize(sl2, acc, wrs2, g2, rs2):
    return pl.pallas_call(
        _final_body,
        out_shape=jax.ShapeDtypeStruct((N, 1), jnp.float32),
    )(sl2, acc, wrs2, g2, rs2)


def kernel(node_attrs, edge_attrs, instruction_batch, distribution,
           node_prop_similarities, relation_similarity,
           weight_node_properties, weight_edge, weight_node_score,
           weight_relation_score, node_indices, edge_batch_indices,
           edge_indices):
    g = node_indices.astype(jnp.int32)
    eb = edge_batch_indices.astype(jnp.int32)
    src = edge_indices[0].astype(jnp.int32)
    dst = edge_indices[1].astype(jnp.int32)

    src3 = src.reshape(NW, NCH1, CH1)
    d = _sc_gather(src3, distribution)

    weT = weight_edge.T
    eb3 = eb.reshape(E // EB, 1, EB)
    d2 = d.reshape(E, 1)
    m = _edge_messages(edge_attrs, weT, instruction_batch, d2, eb3)

    dst3 = dst.reshape(NW, NCH, CH)
    zeros = jnp.zeros((N, D), jnp.float32)
    acc = _sc_scatter(m, dst3, zeros)

    x2 = node_attrs.reshape(N, P * D)
    wT = jnp.transpose(weight_node_properties, (0, 2, 1)).reshape(P * D, D)
    wns2 = weight_node_score.reshape(1, D)
    g3 = g.reshape(N // NB, 1, NB)
    sl = _state_logits(x2, wT, node_prop_similarities,
                       instruction_batch, wns2, g3)

    sl2 = sl.reshape(N, 1)
    g2 = g.reshape(N, 1)
    wrs2 = weight_relation_score.reshape(1, D)
    rs2 = relation_similarity.reshape(1, B)
    out = _finalize(sl2, acc, wrs2, g2, rs2)
    return out.reshape(N)


# speedup vs baseline: 8.8872x; 1.3479x over previous
import functools

import jax
import jax.numpy as jnp
from jax import lax
from jax.experimental import pallas as pl
from jax.experimental.pallas import tpu as pltpu
from jax.experimental.pallas import tpu_sc as plsc

N = 10000
E = 320000
B = 64
P = 4
D = 128

EB = 4000
NB = 1000
NC = 2
NS = 16
NW = NC * NS
EPW = E // NW
CH1 = 80
NCH1 = EPW // CH1
CH = 80
NCH = EPW // CH


def _elu(t):
    return jnp.where(t > 0, t, jnp.exp(jnp.minimum(t, 0.0)) - 1.0)


def _sc_gather(src3, dist):
    mesh = plsc.VectorSubcoreMesh(core_axis_name="c", subcore_axis_name="s")

    @functools.partial(
        pl.kernel,
        mesh=mesh,
        out_type=jax.ShapeDtypeStruct((NW, EPW), jnp.float32),
        scratch_types=[
            pltpu.VMEM((NCH1, CH1), jnp.int32),
            pltpu.VMEM((EPW,), jnp.float32),
            pltpu.SemaphoreType.DMA,
        ],
    )
    def sc_k(src_hbm, dist_hbm, out_hbm, srci, dv, sem):
        wid = lax.axis_index("s") * NC + lax.axis_index("c")
        pltpu.sync_copy(src_hbm.at[wid], srci)

        def fire(j, carry):
            pltpu.async_copy(dist_hbm.at[srci.at[j]],
                             dv.at[pl.ds(j * CH1, CH1)], sem)
            return carry

        lax.fori_loop(0, NCH1, fire, 0)

        def drain(j, carry):
            pltpu.make_async_copy(dist_hbm.at[srci.at[j]],
                                  dv.at[pl.ds(j * CH1, CH1)], sem).wait()
            return carry

        lax.fori_loop(0, NCH1, drain, 0)
        pltpu.sync_copy(dv, out_hbm.at[wid])

    return sc_k(src3, dist)


def _edge_body(xe_ref, we_ref, instr_ref, d_ref, eb_ref, out_ref):
    z = jnp.dot(xe_ref[...].astype(jnp.bfloat16),
                we_ref[...].astype(jnp.bfloat16),
                preferred_element_type=jnp.float32)
    eb = eb_ref[0]
    iota = lax.broadcasted_iota(jnp.int32, (B, 1), 0)
    oh = (iota == eb).astype(jnp.float32)
    ir = lax.dot_general(oh, instr_ref[...],
                         (((0,), (0,)), ((), ())),
                         preferred_element_type=jnp.float32,
                         precision=lax.Precision.HIGHEST)
    dcol = jnp.transpose(d_ref[0], (1, 0))
    out_ref[...] = dcol * _elu(ir * z)


def _edge_messages(edge_attrs, weT, instr, d2, eb3):
    return pl.pallas_call(
        _edge_body,
        grid=(E // EB,),
        in_specs=[
            pl.BlockSpec((EB, D), lambda i: (i, 0)),
            pl.BlockSpec((D, D), lambda i: (0, 0)),
            pl.BlockSpec((B, D), lambda i: (0, 0)),
            pl.BlockSpec((1, 1, EB), lambda i: (i, 0, 0)),
            pl.BlockSpec((1, 1, EB), lambda i: (i, 0, 0)),
        ],
        out_specs=pl.BlockSpec((EB, D), lambda i: (i, 0)),
        out_shape=jax.ShapeDtypeStruct((E, D), jnp.float32),
    )(edge_attrs, weT, instr, d2, eb3)


def _sc_scatter(m, dst3, zeros):
    mesh = plsc.VectorSubcoreMesh(core_axis_name="c", subcore_axis_name="s")

    @functools.partial(
        pl.kernel,
        mesh=mesh,
        out_type=jax.ShapeDtypeStruct((NC, N, D), jnp.float32),
        scratch_types=[
            pltpu.VMEM((NCH, CH), jnp.int32),
            pltpu.VMEM((2, CH, D), jnp.float32),
            pltpu.VMEM_SHARED((N, D), jnp.float32),
            pltpu.SemaphoreType.DMA,
        ],
    )
    def sc_k(m_hbm, dst_hbm, zeros_hbm, out_hbm, dsti, mrow, acc, sem):
        c = lax.axis_index("c")
        sid = lax.axis_index("s")
        wid = sid * NC + c
        base = wid * EPW
        pltpu.sync_copy(dst_hbm.at[wid], dsti)

        @pl.when(sid == 0)
        def _():
            pltpu.sync_copy(zeros_hbm, acc)

        plsc.subcore_barrier()

        for b in range(2):
            pltpu.async_copy(m_hbm.at[pl.ds(base + b * CH, CH)],
                             mrow.at[b], sem)

        def body(t, carry):
            for b in range(2):
                k = 2 * t + b
                pltpu.make_async_copy(m_hbm.at[pl.ds(base + k * CH, CH)],
                                      mrow.at[b], sem).wait()
                pltpu.sync_copy(mrow.at[b], acc.at[dsti.at[k]], add=True)

                @pl.when(k + 2 < NCH)
                def _():
                    pltpu.async_copy(
                        m_hbm.at[pl.ds(base + (k + 2) * CH, CH)],
                        mrow.at[b], sem)
            return carry

        lax.fori_loop(0, NCH // 2, body, 0)
        kt = NCH - 1
        pltpu.make_async_copy(m_hbm.at[pl.ds(base + kt * CH, CH)],
                              mrow.at[0], sem).wait()
        pltpu.sync_copy(mrow.at[0], acc.at[dsti.at[kt]], add=True)
        plsc.subcore_barrier()

        @pl.when(sid == 0)
        def _():
            pltpu.sync_copy(acc, out_hbm.at[c])

    return sc_k(m, dst3, zeros)


def _node_body(x_ref, wT_ref, nps_ref, instr_ref, wns_ref, g_ref, out_ref):
    g = g_ref[0]
    iota = lax.broadcasted_iota(jnp.int32, (B, 1), 0)
    oh = (iota == g).astype(jnp.float32)
    a = lax.dot_general(oh, nps_ref[...],
                        (((0,), (0,)), ((), ())),
                        preferred_element_type=jnp.float32,
                        precision=lax.Precision.HIGHEST)
    ir = lax.dot_general(oh, instr_ref[...],
                         (((0,), (0,)), ((), ())),
                         preferred_element_type=jnp.float32,
                         precision=lax.Precision.HIGHEST)
    acc = jnp.zeros((NB, D), jnp.float32)
    for p in range(P):
        zp = jnp.dot(x_ref[:, p * D:(p + 1) * D].astype(jnp.bfloat16),
                     wT_ref[p * D:(p + 1) * D, :].astype(jnp.bfloat16),
                     preferred_element_type=jnp.float32)
        acc = acc + (a[:, p:p + 1] * ir) * zp
    ns = _elu(acc)
    out_ref[0, 0, :] = jnp.dot(ns.astype(jnp.bfloat16),
                               wns_ref[...].astype(jnp.bfloat16).reshape(D),
                               preferred_element_type=jnp.float32)


def _state_logits(x2, wT, nps, instr, wns2, g3):
    o3 = pl.pallas_call(
        _node_body,
        grid=(N // NB,),
        in_specs=[
            pl.BlockSpec((NB, P * D), lambda i: (i, 0)),
            pl.BlockSpec((P * D, D), lambda i: (0, 0)),
            pl.BlockSpec((B, P), lambda i: (0, 0)),
            pl.BlockSpec((B, D), lambda i: (0, 0)),
            pl.BlockSpec((1, D), lambda i: (0, 0)),
            pl.BlockSpec((1, 1, NB), lambda i: (i, 0, 0)),
        ],
        out_specs=pl.BlockSpec((1, 1, NB), lambda i: (i, 0, 0)),
        out_shape=jax.ShapeDtypeStruct((N // NB, 1, NB), jnp.float32),
    )(x2, wT, nps, instr, wns2, g3)
    return o3.reshape(N)


def _final_body(sl_ref, acc_ref, wrs_ref, g_ref, rs_ref, out_ref):
    ap = acc_ref[0] + acc_ref[1]
    rel = jnp.dot(ap.astype(jnp.bfloat16),
                  wrs_ref[...].astype(jnp.bfloat16).reshape(D, 1),
                  preferred_element_type=jnp.float32)
    g = g_ref[...]
    iota = lax.broadcasted_iota(jnp.int32, (1, B), 1)
    gmask = g == iota
    sl = sl_ref[...]

    def seg_softmax(l):
        mx = jnp.max(jnp.where(gmask, l, -1e30), axis=0, keepdims=True)
        mxg = jnp.sum(jnp.where(gmask, mx, 0.0), axis=1, keepdims=True)
        ex = jnp.exp(l - mxg)
        sm = jnp.sum(jnp.where(gmask, ex, 0.0), axis=0, keepdims=True)
        smg = jnp.sum(jnp.where(gmask, sm, 0.0), axis=1, keepdims=True)
        return ex / smg

    ps = seg_softmax(sl)
    pr = seg_softmax(rel)
    rsg = jnp.sum(jnp.where(gmask, rs_ref[...], 0.0), axis=1, keepdims=True)
    out_ref[...] = rsg * pr + (1.0 - rsg) * ps


def _finalize(sl2, acc, wrs2, g2, rs2):
    return pl.pallas_call(
        _final_body,
        out_shape=jax.ShapeDtypeStruct((N, 1), jnp.float32),
    )(sl2, acc, wrs2, g2, rs2)


def kernel(node_attrs, edge_attrs, instruction_batch, distribution,
           node_prop_similarities, relation_similarity,
           weight_node_properties, weight_edge, weight_node_score,
           weight_relation_score, node_indices, edge_batch_indices,
           edge_indices):
    g = node_indices.astype(jnp.int32)
    eb = edge_batch_indices.astype(jnp.int32)
    src = edge_indices[0].astype(jnp.int32)
    dst = edge_indices[1].astype(jnp.int32)

    src3 = src.reshape(NW, NCH1, CH1)
    d = _sc_gather(src3, distribution)

    weT = weight_edge.T
    eb3 = eb.reshape(E // EB, 1, EB)
    d3 = d.reshape(E // EB, 1, EB)
    m = _edge_messages(edge_attrs, weT, instruction_batch, d3, eb3)

    dst3 = dst.reshape(NW, NCH, CH)
    zeros = jnp.zeros((N, D), jnp.float32)
    acc = _sc_scatter(m, dst3, zeros)

    x2 = node_attrs.reshape(N, P * D)
    wT = jnp.transpose(weight_node_properties, (0, 2, 1)).reshape(P * D, D)
    wns2 = weight_node_score.reshape(1, D)
    g3 = g.reshape(N // NB, 1, NB)
    sl = _state_logits(x2, wT, node_prop_similarities,
                       instruction_batch, wns2, g3)

    sl2 = sl.reshape(N, 1)
    g2 = g.reshape(N, 1)
    wrs2 = weight_relation_score.reshape(1, D)
    rs2 = relation_similarity.reshape(1, B)
    out = _finalize(sl2, acc, wrs2, g2, rs2)
    return out.reshape(N)


# speedup vs baseline: 8.8880x; 1.0001x over previous
import functools

import jax
import jax.numpy as jnp
from jax import lax
from jax.experimental import pallas as pl
from jax.experimental.pallas import tpu as pltpu
from jax.experimental.pallas import tpu_sc as plsc

N = 10000
E = 320000
B = 64
P = 4
D = 128

EB = 4000
NB = 1000
NC = 2
NS = 16
NW = NC * NS
EPW = E // NW
CH1 = 80
NCH1 = EPW // CH1
CH = 80
NCH = EPW // CH


def _elu(t):
    return jnp.where(t > 0, t, jnp.exp(jnp.minimum(t, 0.0)) - 1.0)


def _sc_gather(src3, dist):
    mesh = plsc.VectorSubcoreMesh(core_axis_name="c", subcore_axis_name="s")

    @functools.partial(
        pl.kernel,
        mesh=mesh,
        out_type=jax.ShapeDtypeStruct((NW, EPW), jnp.float32),
        scratch_types=[
            pltpu.VMEM((NCH1, CH1), jnp.int32),
            pltpu.VMEM((EPW,), jnp.float32),
            pltpu.SemaphoreType.DMA,
        ],
    )
    def sc_k(src_hbm, dist_hbm, out_hbm, srci, dv, sem):
        wid = lax.axis_index("s") * NC + lax.axis_index("c")
        pltpu.sync_copy(src_hbm.at[wid], srci)

        def fire(j, carry):
            pltpu.async_copy(dist_hbm.at[srci.at[j]],
                             dv.at[pl.ds(j * CH1, CH1)], sem)
            return carry

        lax.fori_loop(0, NCH1, fire, 0)

        def drain(j, carry):
            pltpu.make_async_copy(dist_hbm.at[srci.at[j]],
                                  dv.at[pl.ds(j * CH1, CH1)], sem).wait()
            return carry

        lax.fori_loop(0, NCH1, drain, 0)
        pltpu.sync_copy(dv, out_hbm.at[wid])

    return sc_k(src3, dist)


def _edge_body(xe_ref, we_ref, instr_ref, d_ref, eb_ref, out_ref):
    z = jnp.dot(xe_ref[...].astype(jnp.bfloat16),
                we_ref[...].astype(jnp.bfloat16),
                preferred_element_type=jnp.float32)
    eb = eb_ref[0]
    iota = lax.broadcasted_iota(jnp.int32, (B, 1), 0)
    oh = (iota == eb).astype(jnp.float32)
    ir = lax.dot_general(oh, instr_ref[...],
                         (((0,), (0,)), ((), ())),
                         preferred_element_type=jnp.float32,
                         precision=lax.Precision.HIGHEST)
    dcol = jnp.transpose(d_ref[0], (1, 0))
    out_ref[...] = dcol * _elu(ir * z)


def _edge_messages(edge_attrs, weT, instr, d2, eb3):
    return pl.pallas_call(
        _edge_body,
        grid=(E // EB,),
        in_specs=[
            pl.BlockSpec((EB, D), lambda i: (i, 0)),
            pl.BlockSpec((D, D), lambda i: (0, 0)),
            pl.BlockSpec((B, D), lambda i: (0, 0)),
            pl.BlockSpec((1, 1, EB), lambda i: (i, 0, 0)),
            pl.BlockSpec((1, 1, EB), lambda i: (i, 0, 0)),
        ],
        out_specs=pl.BlockSpec((EB, D), lambda i: (i, 0)),
        out_shape=jax.ShapeDtypeStruct((E, D), jnp.float32),
    )(edge_attrs, weT, instr, d2, eb3)


def _sc_scatter(m, dst3, zeros):
    mesh = plsc.VectorSubcoreMesh(core_axis_name="c", subcore_axis_name="s")

    @functools.partial(
        pl.kernel,
        mesh=mesh,
        out_type=jax.ShapeDtypeStruct((NC, N, D), jnp.float32),
        scratch_types=[
            pltpu.VMEM((NCH, CH), jnp.int32),
            pltpu.VMEM((2, CH, D), jnp.float32),
            pltpu.VMEM_SHARED((N, D), jnp.float32),
            pltpu.SemaphoreType.DMA,
        ],
    )
    def sc_k(m_hbm, dst_hbm, zeros_hbm, out_hbm, dsti, mrow, acc, sem):
        c = lax.axis_index("c")
        sid = lax.axis_index("s")
        wid = sid * NC + c
        base = wid * EPW
        pltpu.sync_copy(dst_hbm.at[wid], dsti)

        @pl.when(sid == 0)
        def _():
            pltpu.sync_copy(zeros_hbm, acc)

        plsc.subcore_barrier()

        for b in range(2):
            pltpu.async_copy(m_hbm.at[pl.ds(base + b * CH, CH)],
                             mrow.at[b], sem)

        def body(t, carry):
            for b in range(2):
                k = 2 * t + b
                pltpu.make_async_copy(m_hbm.at[pl.ds(base + k * CH, CH)],
                                      mrow.at[b], sem).wait()
                pltpu.sync_copy(mrow.at[b], acc.at[dsti.at[k]], add=True)

                @pl.when(k + 2 < NCH)
                def _():
                    pltpu.async_copy(
                        m_hbm.at[pl.ds(base + (k + 2) * CH, CH)],
                        mrow.at[b], sem)
            return carry

        lax.fori_loop(0, NCH // 2, body, 0)
        kt = NCH - 1
        pltpu.make_async_copy(m_hbm.at[pl.ds(base + kt * CH, CH)],
                              mrow.at[0], sem).wait()
        pltpu.sync_copy(mrow.at[0], acc.at[dsti.at[kt]], add=True)
        plsc.subcore_barrier()

        @pl.when(sid == 0)
        def _():
            pltpu.sync_copy(acc, out_hbm.at[c])

    return sc_k(m, dst3, zeros)


def _node_body(x_ref, wT_ref, nps_ref, instr_ref, wns_ref, g_ref, out_ref):
    g = g_ref[0]
    iota = lax.broadcasted_iota(jnp.int32, (B, 1), 0)
    oh = (iota == g).astype(jnp.float32)
    a = lax.dot_general(oh, nps_ref[...],
                        (((0,), (0,)), ((), ())),
                        preferred_element_type=jnp.float32,
                        precision=lax.Precision.HIGHEST)
    ir = lax.dot_general(oh, instr_ref[...],
                         (((0,), (0,)), ((), ())),
                         preferred_element_type=jnp.float32,
                         precision=lax.Precision.HIGHEST)
    acc = jnp.zeros((NB, D), jnp.float32)
    for p in range(P):
        zp = jnp.dot(x_ref[:, p * D:(p + 1) * D].astype(jnp.bfloat16),
                     wT_ref[p * D:(p + 1) * D, :].astype(jnp.bfloat16),
                     preferred_element_type=jnp.float32)
        acc = acc + (a[:, p:p + 1] * ir) * zp
    ns = _elu(acc)
    out_ref[0, 0, :] = jnp.dot(ns.astype(jnp.bfloat16),
                               wns_ref[...].astype(jnp.bfloat16).reshape(D),
                               preferred_element_type=jnp.float32)


def _state_logits(x2, wT, nps, instr, wns2, g3):
    o3 = pl.pallas_call(
        _node_body,
        grid=(N // NB,),
        in_specs=[
            pl.BlockSpec((NB, P * D), lambda i: (i, 0)),
            pl.BlockSpec((P * D, D), lambda i: (0, 0)),
            pl.BlockSpec((B, P), lambda i: (0, 0)),
            pl.BlockSpec((B, D), lambda i: (0, 0)),
            pl.BlockSpec((1, D), lambda i: (0, 0)),
            pl.BlockSpec((1, 1, NB), lambda i: (i, 0, 0)),
        ],
        out_specs=pl.BlockSpec((1, 1, NB), lambda i: (i, 0, 0)),
        out_shape=jax.ShapeDtypeStruct((N // NB, 1, NB), jnp.float32),
    )(x2, wT, nps, instr, wns2, g3)
    return o3.reshape(N)


def _final_body(sl_ref, acc_ref, wrs_ref, g_ref, rs_ref, out_ref):
    ap = acc_ref[0] + acc_ref[1]
    rel = jnp.dot(ap.astype(jnp.bfloat16),
                  wrs_ref[...].astype(jnp.bfloat16).reshape(D, 1),
                  preferred_element_type=jnp.float32)
    g = g_ref[...]
    iota = lax.broadcasted_iota(jnp.int32, (1, B), 1)
    gmask = g == iota
    sl = sl_ref[...]

    def seg_softmax(l):
        mx = jnp.max(jnp.where(gmask, l, -1e30), axis=0, keepdims=True)
        mxg = jnp.sum(jnp.where(gmask, mx, 0.0), axis=1, keepdims=True)
        ex = jnp.exp(l - mxg)
        sm = jnp.sum(jnp.where(gmask, ex, 0.0), axis=0, keepdims=True)
        smg = jnp.sum(jnp.where(gmask, sm, 0.0), axis=1, keepdims=True)
        return ex / smg

    ps = seg_softmax(sl)
    pr = seg_softmax(rel)
    rsg = jnp.sum(jnp.where(gmask, rs_ref[...], 0.0), axis=1, keepdims=True)
    out_ref[...] = rsg * pr + (1.0 - rsg) * ps


def _finalize(sl2, acc, wrs2, g2, rs2):
    return pl.pallas_call(
        _final_body,
        out_shape=jax.ShapeDtypeStruct((N, 1), jnp.float32),
    )(sl2, acc, wrs2, g2, rs2)


def kernel(node_attrs, edge_attrs, instruction_batch, distribution,
           node_prop_similarities, relation_similarity,
           weight_node_properties, weight_edge, weight_node_score,
           weight_relation_score, node_indices, edge_batch_indices,
           edge_indices):
    g = node_indices.astype(jnp.int32)
    eb = edge_batch_indices.astype(jnp.int32)
    src = edge_indices[0].astype(jnp.int32)
    dst = edge_indices[1].astype(jnp.int32)

    x2 = node_attrs.reshape(N, P * D)
    wT = jnp.transpose(weight_node_properties, (0, 2, 1)).reshape(P * D, D)
    wns2 = weight_node_score.reshape(1, D)
    g3 = g.reshape(N // NB, 1, NB)
    sl = _state_logits(x2, wT, node_prop_similarities,
                       instruction_batch, wns2, g3)

    src3 = src.reshape(NW, NCH1, CH1)
    d = _sc_gather(src3, distribution)

    weT = weight_edge.T
    eb3 = eb.reshape(E // EB, 1, EB)
    d3 = d.reshape(E // EB, 1, EB)
    m = _edge_messages(edge_attrs, weT, instruction_batch, d3, eb3)

    dst3 = dst.reshape(NW, NCH, CH)
    zeros = jnp.zeros((N, D), jnp.float32)
    acc = _sc_scatter(m, dst3, zeros)

    sl2 = sl.reshape(N, 1)
    g2 = g.reshape(N, 1)
    wrs2 = weight_relation_score.reshape(1, D)
    rs2 = relation_similarity.reshape(1, B)
    out = _finalize(sl2, acc, wrs2, g2, rs2)
    return out.reshape(N)


# speedup vs baseline: 8.9866x; 1.0111x over previous
import functools

import jax
import jax.numpy as jnp
from jax import lax
from jax.experimental import pallas as pl
from jax.experimental.pallas import tpu as pltpu
from jax.experimental.pallas import tpu_sc as plsc

N = 10000
E = 320000
B = 64
P = 4
D = 128

EB = 4000
NB = 1000
NC = 2
NS = 16
NW = NC * NS
EPW = E // NW
CH1 = 80
NCH1 = EPW // CH1
CH = 80
NCH = EPW // CH


def _elu(t):
    return jnp.where(t > 0, t, jnp.exp(t) - 1.0)


def _sc_gather(src3, dist):
    mesh = plsc.VectorSubcoreMesh(core_axis_name="c", subcore_axis_name="s")

    @functools.partial(
        pl.kernel,
        mesh=mesh,
        out_type=jax.ShapeDtypeStruct((NW, EPW), jnp.float32),
        scratch_types=[
            pltpu.VMEM((NCH1, CH1), jnp.int32),
            pltpu.VMEM((EPW,), jnp.float32),
            pltpu.SemaphoreType.DMA,
        ],
    )
    def sc_k(src_hbm, dist_hbm, out_hbm, srci, dv, sem):
        wid = lax.axis_index("s") * NC + lax.axis_index("c")
        pltpu.sync_copy(src_hbm.at[wid], srci)

        def fire(j, carry):
            pltpu.async_copy(dist_hbm.at[srci.at[j]],
                             dv.at[pl.ds(j * CH1, CH1)], sem)
            return carry

        lax.fori_loop(0, NCH1, fire, 0)

        def drain(j, carry):
            pltpu.make_async_copy(dist_hbm.at[srci.at[j]],
                                  dv.at[pl.ds(j * CH1, CH1)], sem).wait()
            return carry

        lax.fori_loop(0, NCH1, drain, 0)
        pltpu.sync_copy(dv, out_hbm.at[wid])

    return sc_k(src3, dist)


def _edge_body(xe_ref, we_ref, instr_ref, d_ref, eb_ref, out_ref):
    z = jnp.dot(xe_ref[...].astype(jnp.bfloat16),
                we_ref[...].astype(jnp.bfloat16),
                preferred_element_type=jnp.float32)
    eb = eb_ref[0]
    iota = lax.broadcasted_iota(jnp.int32, (B, 1), 0)
    oh = (iota == eb).astype(jnp.float32)
    ir = lax.dot_general(oh, instr_ref[...],
                         (((0,), (0,)), ((), ())),
                         preferred_element_type=jnp.float32,
                         precision=lax.Precision.HIGHEST)
    dcol = jnp.transpose(d_ref[0], (1, 0))
    out_ref[...] = dcol * _elu(ir * z)


def _edge_messages(edge_attrs, weT, instr, d2, eb3):
    return pl.pallas_call(
        _edge_body,
        grid=(E // EB,),
        in_specs=[
            pl.BlockSpec((EB, D), lambda i: (i, 0)),
            pl.BlockSpec((D, D), lambda i: (0, 0)),
            pl.BlockSpec((B, D), lambda i: (0, 0)),
            pl.BlockSpec((1, 1, EB), lambda i: (i, 0, 0)),
            pl.BlockSpec((1, 1, EB), lambda i: (i, 0, 0)),
        ],
        out_specs=pl.BlockSpec((EB, D), lambda i: (i, 0)),
        out_shape=jax.ShapeDtypeStruct((E, D), jnp.float32),
    )(edge_attrs, weT, instr, d2, eb3)


def _sc_scatter(m, dst3, zeros):
    mesh = plsc.VectorSubcoreMesh(core_axis_name="c", subcore_axis_name="s")

    @functools.partial(
        pl.kernel,
        mesh=mesh,
        out_type=jax.ShapeDtypeStruct((NC, N, D), jnp.float32),
        scratch_types=[
            pltpu.VMEM((NCH, CH), jnp.int32),
            pltpu.VMEM((2, CH, D), jnp.float32),
            pltpu.VMEM_SHARED((N, D), jnp.float32),
            pltpu.SemaphoreType.DMA,
        ],
    )
    def sc_k(m_hbm, dst_hbm, zeros_hbm, out_hbm, dsti, mrow, acc, sem):
        c = lax.axis_index("c")
        sid = lax.axis_index("s")
        wid = sid * NC + c
        base = wid * EPW
        pltpu.sync_copy(dst_hbm.at[wid], dsti)

        @pl.when(sid == 0)
        def _():
            pltpu.sync_copy(zeros_hbm, acc)

        plsc.subcore_barrier()

        for b in range(2):
            pltpu.async_copy(m_hbm.at[pl.ds(base + b * CH, CH)],
                             mrow.at[b], sem)

        def body(t, carry):
            for b in range(2):
                k = 2 * t + b
                pltpu.make_async_copy(m_hbm.at[pl.ds(base + k * CH, CH)],
                                      mrow.at[b], sem).wait()
                pltpu.sync_copy(mrow.at[b], acc.at[dsti.at[k]], add=True)

                @pl.when(k + 2 < NCH)
                def _():
                    pltpu.async_copy(
                        m_hbm.at[pl.ds(base + (k + 2) * CH, CH)],
                        mrow.at[b], sem)
            return carry

        lax.fori_loop(0, NCH // 2, body, 0)
        kt = NCH - 1
        pltpu.make_async_copy(m_hbm.at[pl.ds(base + kt * CH, CH)],
                              mrow.at[0], sem).wait()
        pltpu.sync_copy(mrow.at[0], acc.at[dsti.at[kt]], add=True)
        plsc.subcore_barrier()

        @pl.when(sid == 0)
        def _():
            pltpu.sync_copy(acc, out_hbm.at[c])

    return sc_k(m, dst3, zeros)


def _node_body(x_ref, wT_ref, nps_ref, instr_ref, wns_ref, g_ref, out_ref):
    g = g_ref[0]
    iota = lax.broadcasted_iota(jnp.int32, (B, 1), 0)
    oh = (iota == g).astype(jnp.float32)
    a = lax.dot_general(oh, nps_ref[...],
                        (((0,), (0,)), ((), ())),
                        preferred_element_type=jnp.float32,
                        precision=lax.Precision.HIGHEST)
    ir = lax.dot_general(oh, instr_ref[...],
                         (((0,), (0,)), ((), ())),
                         preferred_element_type=jnp.float32,
                         precision=lax.Precision.HIGHEST)
    acc = jnp.zeros((NB, D), jnp.float32)
    for p in range(P):
        zp = jnp.dot(x_ref[:, p * D:(p + 1) * D].astype(jnp.bfloat16),
                     wT_ref[p * D:(p + 1) * D, :].astype(jnp.bfloat16),
                     preferred_element_type=jnp.float32)
        acc = acc + (a[:, p:p + 1] * ir) * zp
    ns = _elu(acc)
    out_ref[0, 0, :] = jnp.dot(ns.astype(jnp.bfloat16),
                               wns_ref[...].astype(jnp.bfloat16).reshape(D),
                               preferred_element_type=jnp.float32)


def _state_logits(x2, wT, nps, instr, wns2, g3):
    o3 = pl.pallas_call(
        _node_body,
        grid=(N // NB,),
        in_specs=[
            pl.BlockSpec((NB, P * D), lambda i: (i, 0)),
            pl.BlockSpec((P * D, D), lambda i: (0, 0)),
            pl.BlockSpec((B, P), lambda i: (0, 0)),
            pl.BlockSpec((B, D), lambda i: (0, 0)),
            pl.BlockSpec((1, D), lambda i: (0, 0)),
            pl.BlockSpec((1, 1, NB), lambda i: (i, 0, 0)),
        ],
        out_specs=pl.BlockSpec((1, 1, NB), lambda i: (i, 0, 0)),
        out_shape=jax.ShapeDtypeStruct((N // NB, 1, NB), jnp.float32),
    )(x2, wT, nps, instr, wns2, g3)
    return o3.reshape(N)


def _final_body(sl_ref, acc_ref, wrs_ref, g_ref, rs_ref, out_ref):
    ap = acc_ref[0] + acc_ref[1]
    rel = jnp.dot(ap.astype(jnp.bfloat16),
                  wrs_ref[...].astype(jnp.bfloat16).reshape(D, 1),
                  preferred_element_type=jnp.float32)
    g = g_ref[...]
    iota = lax.broadcasted_iota(jnp.int32, (1, B), 1)
    gmask = g == iota
    sl = sl_ref[...]

    def seg_softmax(l):
        mx = jnp.max(jnp.where(gmask, l, -1e30), axis=0, keepdims=True)
        mxg = jnp.sum(jnp.where(gmask, mx, 0.0), axis=1, keepdims=True)
        ex = jnp.exp(l - mxg)
        sm = jnp.sum(jnp.where(gmask, ex, 0.0), axis=0, keepdims=True)
        smg = jnp.sum(jnp.where(gmask, sm, 0.0), axis=1, keepdims=True)
        return ex / smg

    ps = seg_softmax(sl)
    pr = seg_softmax(rel)
    rsg = jnp.sum(jnp.where(gmask, rs_ref[...], 0.0), axis=1, keepdims=True)
    out_ref[...] = rsg * pr + (1.0 - rsg) * ps


def _finalize(sl2, acc, wrs2, g2, rs2):
    return pl.pallas_call(
        _final_body,
        out_shape=jax.ShapeDtypeStruct((N, 1), jnp.float32),
    )(sl2, acc, wrs2, g2, rs2)


def kernel(node_attrs, edge_attrs, instruction_batch, distribution,
           node_prop_similarities, relation_similarity,
           weight_node_properties, weight_edge, weight_node_score,
           weight_relation_score, node_indices, edge_batch_indices,
           edge_indices):
    g = node_indices.astype(jnp.int32)
    eb = edge_batch_indices.astype(jnp.int32)
    src = edge_indices[0].astype(jnp.int32)
    dst = edge_indices[1].astype(jnp.int32)

    x2 = node_attrs.reshape(N, P * D)
    wT = jnp.transpose(weight_node_properties, (0, 2, 1)).reshape(P * D, D)
    wns2 = weight_node_score.reshape(1, D)
    g3 = g.reshape(N // NB, 1, NB)
    sl = _state_logits(x2, wT, node_prop_similarities,
                       instruction_batch, wns2, g3)

    src3 = src.reshape(NW, NCH1, CH1)
    d = _sc_gather(src3, distribution)

    weT = weight_edge.T
    eb3 = eb.reshape(E // EB, 1, EB)
    d3 = d.reshape(E // EB, 1, EB)
    m = _edge_messages(edge_attrs, weT, instruction_batch, d3, eb3)

    dst3 = dst.reshape(NW, NCH, CH)
    zeros = jnp.zeros((N, D), jnp.float32)
    acc = _sc_scatter(m, dst3, zeros)

    sl2 = sl.reshape(N, 1)
    g2 = g.reshape(N, 1)
    wrs2 = weight_relation_score.reshape(1, D)
    rs2 = relation_similarity.reshape(1, B)
    out = _finalize(sl2, acc, wrs2, g2, rs2)
    return out.reshape(N)
